# baseline (speedup 1.0000x reference)
import jax
import jax.numpy as jnp
from jax import lax
from jax.experimental import pallas as pl
from jax.experimental.pallas import tpu as pltpu
from jax.experimental.pallas import tpu_sc as plsc

_BF = jnp.bfloat16


def _mm(a, b):
    return lax.dot_general(a.astype(_BF), b, (((1,), (0,)), ((), ())),
                           preferred_element_type=jnp.float32)


def _shift_down(a):
    return jnp.concatenate([jnp.zeros((1, a.shape[1]), a.dtype), a[:-1]], axis=0)


def _shift_up(a):
    return jnp.concatenate([a[1:], jnp.zeros((1, a.shape[1]), a.dtype)], axis=0)


def _enc_layer(phases, w, b):
    m = len(phases)
    out = []
    for q in range(m // 2):
        lo = 2 * q - 1
        a = _shift_down(phases[m - 1]) if lo < 0 else phases[lo]
        cat = jnp.concatenate([a, phases[2 * q], phases[2 * q + 1]], axis=1)
        out.append(jax.nn.relu(_mm(cat, w) + b))
    return out


def _dec_layer(phases, w1, w20, b):
    m = len(phases)
    out = []
    for i in range(m):
        out.append(jax.nn.relu(_mm(phases[i], w1) + b))
        nxt = phases[i + 1] if i < m - 1 else _shift_up(phases[0])
        cat = jnp.concatenate([phases[i], nxt], axis=1)
        out.append(jax.nn.relu(_mm(cat, w20) + b))
    return out


def _encoder_body(xp_ref, wpt_ref, bp_ref, w1_ref, b1_ref, w2_ref, b2_ref,
                  w3_ref, b3_ref, wqt_ref, bq_ref, cbt_ref, cbn2_ref,
                  z_ref, idx_ref):
    x = xp_ref[0]
    wpt = wpt_ref[...]
    bp = bp_ref[...]
    h = [_mm(x[p], wpt) + bp for p in range(8)]

    y = _enc_layer(h, w1_ref[...], b1_ref[...])
    y = _enc_layer(y, w2_ref[...], b2_ref[...])
    y = _enc_layer(y, w3_ref[...], b3_ref[...])
    z = _mm(y[0], wqt_ref[...]) + bq_ref[...]
    z_ref[0] = z

    zc = _mm(z, cbt_ref[...])
    zn2 = jnp.sum(z * z, axis=1, keepdims=True)
    d2 = zn2 - 2.0 * zc + cbn2_ref[...]
    d2 = jnp.maximum(d2, 0.0)
    mn = jnp.min(d2, axis=1, keepdims=True)
    iota = lax.broadcasted_iota(jnp.int32, d2.shape, 1)
    idx = jnp.min(jnp.where(d2 == mn, iota, d2.shape[1]), axis=1)
    idx_ref[0, 0] = idx


def _decoder_body(idx_ref, cbh_ref, cb_ref, w1a_ref, w1b_ref, b1_ref,
                  w2a_ref, w2b_ref, b2_ref, w3a_ref, w3b_ref, b3_ref,
                  wot_ref, bo_ref, rec_ref, zq_ref, cbl_ref):
    b = pl.program_id(0)

    @pl.when(b == 0)
    def _():
        cbl_ref[...] = (cb_ref[...] - cbh_ref[...].astype(jnp.float32)).astype(_BF)

    idx = idx_ref[0, 0]
    k = cbh_ref.shape[0]
    iota = lax.broadcasted_iota(jnp.int32, (idx.shape[0], k), 1)
    oh = (iota == idx[:, None]).astype(_BF)
    ghi = _mm(oh, cbh_ref[...])
    zq_ref[0] = ghi + _mm(oh, cbl_ref[...])
    r = _dec_layer([ghi], w1a_ref[...], w1b_ref[...], b1_ref[...])
    r = _dec_layer(r, w2a_ref[...], w2b_ref[...], b2_ref[...])
    r = _dec_layer(r, w3a_ref[...], w3b_ref[...], b3_ref[...])
    wot = wot_ref[...]
    bo = bo_ref[...]
    for j in range(8):
        rec_ref[0, j] = _mm(r[j], wot) + bo


def _sc_gather(codebook, indices):
    n = indices.shape[0]
    d = codebook.shape[1]
    nw = 32
    bpw = n // nw
    nfly = 4
    ck = bpw // nfly
    mesh = plsc.VectorSubcoreMesh(core_axis_name="c", subcore_axis_name="s")

    @pl.kernel(out_type=jax.ShapeDtypeStruct((n, d), codebook.dtype),
               mesh=mesh,
               scratch_types=[pltpu.VMEM((bpw,), jnp.int32),
                              pltpu.VMEM((bpw, d), codebook.dtype),
                              pltpu.SemaphoreType.DMA])
    def gather_kernel(cb_hbm, i_hbm, o_hbm, idx_v, rows_v, sem):
        wid = lax.axis_index("s") * 2 + lax.axis_index("c")
        base = wid * bpw
        pltpu.sync_copy(i_hbm.at[pl.ds(base, bpw)], idx_v)
        copies = [pltpu.async_copy(cb_hbm.at[idx_v.at[pl.ds(j * ck, ck)]],
                                   rows_v.at[pl.ds(j * ck, ck)], sem)
                  for j in range(nfly)]
        for c in copies:
            c.wait()
        pltpu.sync_copy(rows_v, o_hbm.at[pl.ds(base, bpw)])

    return gather_kernel(codebook, indices)


def _encode(xp, wpt, bp, w1, b1, w2, b2, w3, b3, wqt, bq, cbt, cbn2,
            interpret=False):
    B, P, U, F = xp.shape
    H = wpt.shape[1]
    K = cbt.shape[1]
    D = wqt.shape[1]
    full = lambda *shape: pl.BlockSpec(shape, lambda b: (0,) * len(shape))
    return pl.pallas_call(
        _encoder_body,
        grid=(B,),
        in_specs=[
            pl.BlockSpec((1, P, U, F), lambda b: (b, 0, 0, 0)),
            full(F, H), full(1, H),
            full(3 * H, H), full(1, H),
            full(3 * H, H), full(1, H),
            full(3 * H, H), full(1, H),
            full(H, D), full(1, D),
            full(D, K), full(1, K),
        ],
        out_specs=[
            pl.BlockSpec((1, U, D), lambda b: (b, 0, 0)),
            pl.BlockSpec((1, 1, U), lambda b: (b, 0, 0)),
        ],
        out_shape=[
            jax.ShapeDtypeStruct((B, U, D), jnp.float32),
            jax.ShapeDtypeStruct((B, 1, U), jnp.int32),
        ],
        interpret=interpret,
    )(xp, wpt, bp, w1, b1, w2, b2, w3, b3, wqt, bq, cbt, cbn2)


def _decode(idx, cbh, cb, w1a, w1b, b1, w2a, w2b, b2, w3a, w3b, b3, wot, bo,
            interpret=False):
    B, _, U = idx.shape
    K, D = cbh.shape
    H = w1b.shape[1]
    F = wot.shape[1]
    full = lambda *shape: pl.BlockSpec(shape, lambda b: (0,) * len(shape))
    return pl.pallas_call(
        _decoder_body,
        grid=(B,),
        in_specs=[
            pl.BlockSpec((1, 1, U), lambda b: (b, 0, 0)),
            full(K, D), full(K, D),
            full(D, H), full(2 * D, H), full(1, H),
            full(H, H), full(2 * H, H), full(1, H),
            full(H, H), full(2 * H, H), full(1, H),
            full(H, F), full(1, F),
        ],
        out_specs=[
            pl.BlockSpec((1, 8, U, F), lambda b: (b, 0, 0, 0)),
            pl.BlockSpec((1, U, D), lambda b: (b, 0, 0)),
        ],
        out_shape=[
            jax.ShapeDtypeStruct((B, 8, U, F), jnp.float32),
            jax.ShapeDtypeStruct((B, U, D), jnp.float32),
        ],
        scratch_shapes=[pltpu.VMEM((K, D), _BF)],
        interpret=interpret,
    )(idx, cbh, cb, w1a, w1b, b1, w2a, w2b, b2, w3a, w3b, b3, wot, bo)


def kernel(x, Wp, bp, We1, be1, We2, be2, We3, be3, Wq, bq, codebook,
           Wd1, bd1, Wd2, bd2, Wd3, bd3, Wo, bo):
    B, F, T = x.shape
    H = Wp.shape[0]
    K, D = codebook.shape
    P = 8
    U = T // P

    xp = x.transpose(0, 2, 1).reshape(B, U, P, F).transpose(0, 2, 1, 3)
    wpt = Wp.T.astype(_BF)
    enc_w = lambda w: w.transpose(2, 1, 0).reshape(3 * H, H).astype(_BF)
    w1t, w2t, w3t = enc_w(We1), enc_w(We2), enc_w(We3)
    wqt = Wq[:, :, 0].T.astype(_BF)
    cbt = codebook.T.astype(_BF)
    cbn2 = jnp.sum(codebook ** 2, axis=1).reshape(1, K)
    dec_w1 = lambda w: w[:, :, 1].astype(_BF)
    dec_w20 = lambda w: jnp.concatenate([w[:, :, 2], w[:, :, 0]], axis=0).astype(_BF)
    wot = Wo.T.astype(_BF)
    r1 = lambda v: v.reshape(1, -1)

    cbh = codebook.astype(_BF)

    z_rows, idx = _encode(xp, wpt, r1(bp), w1t, r1(be1), w2t, r1(be2),
                          w3t, r1(be3), wqt, r1(bq), cbt, cbn2)
    indices = idx.reshape(B * U)
    z = z_rows.transpose(0, 2, 1)

    rec, zq_rows = _decode(idx, cbh, codebook,
                           dec_w1(Wd1), dec_w20(Wd1), r1(bd1),
                           dec_w1(Wd2), dec_w20(Wd2), r1(bd2),
                           dec_w1(Wd3), dec_w20(Wd3), r1(bd3), wot, r1(bo))
    z_q = zq_rows.transpose(0, 2, 1)
    recon = rec.transpose(0, 3, 2, 1).reshape(B, F, T)
    return (recon, z, z_q, indices)


# speedup vs baseline: 1.0892x; 1.0892x over previous
import jax
import jax.numpy as jnp
from jax import lax
from jax.experimental import pallas as pl
from jax.experimental.pallas import tpu as pltpu

_BF = jnp.bfloat16


def _mm(a, b):
    return lax.dot_general(a.astype(_BF), b, (((1,), (0,)), ((), ())),
                           preferred_element_type=jnp.float32)


def _mm_nt(a, b):
    return lax.dot_general(a.astype(_BF), b.astype(_BF), (((1,), (1,)), ((), ())),
                           preferred_element_type=jnp.float32)


def _shift_down(a):
    return jnp.concatenate([jnp.zeros((1, a.shape[1]), a.dtype), a[:-1]], axis=0)


def _shift_up(a):
    return jnp.concatenate([a[1:], jnp.zeros((1, a.shape[1]), a.dtype)], axis=0)


def _enc_layer(phases, w, b):
    m = len(phases)
    out = []
    for q in range(m // 2):
        lo = 2 * q - 1
        a = _shift_down(phases[m - 1]) if lo < 0 else phases[lo]
        cat = jnp.concatenate([a, phases[2 * q], phases[2 * q + 1]], axis=1)
        out.append(jax.nn.relu(_mm(cat, w) + b))
    return out


def _dec_layer(phases, w1, w20, b):
    m = len(phases)
    out = []
    for i in range(m):
        out.append(jax.nn.relu(_mm(phases[i], w1) + b))
        nxt = phases[i + 1] if i < m - 1 else _shift_up(phases[0])
        cat = jnp.concatenate([phases[i], nxt], axis=1)
        out.append(jax.nn.relu(_mm(cat, w20) + b))
    return out


def _body(x_ref, wpt_ref, bp_ref, w1_ref, b1_ref, w2_ref, b2_ref,
          w3_ref, b3_ref, wqt_ref, bq_ref, cbt_ref, cbn2_ref,
          cbh_ref, cb_ref, d1a_ref, d1b_ref, bd1_ref, d2a_ref, d2b_ref,
          bd2_ref, d3a_ref, d3b_ref, bd3_ref, wo_ref, bo_ref,
          rec_ref, z_ref, zq_ref, idx_ref, cbl_ref):
    b = pl.program_id(0)

    @pl.when(b == 0)
    def _():
        cbl_ref[...] = (cb_ref[...] - cbh_ref[...].astype(jnp.float32)).astype(_BF)

    x = x_ref[0]
    wpt = wpt_ref[...]
    bp = bp_ref[...]
    h = [_mm(x[p], wpt) + bp for p in range(8)]

    y = _enc_layer(h, w1_ref[...], b1_ref[...])
    y = _enc_layer(y, w2_ref[...], b2_ref[...])
    y = _enc_layer(y, w3_ref[...], b3_ref[...])
    z = _mm(y[0], wqt_ref[...]) + bq_ref[...]
    z_ref[0] = z.T

    zc = _mm(z, cbt_ref[...])
    zn2 = jnp.sum(z * z, axis=1, keepdims=True)
    d2 = zn2 - 2.0 * zc + cbn2_ref[...]
    d2 = jnp.maximum(d2, 0.0)
    mn = jnp.min(d2, axis=1, keepdims=True)
    iota = lax.broadcasted_iota(jnp.int32, d2.shape, 1)
    idx = jnp.min(jnp.where(d2 == mn, iota, d2.shape[1]), axis=1)
    idx_ref[0, 0] = idx

    oh = (iota == idx[:, None]).astype(_BF)
    ghi = _mm(oh, cbh_ref[...])
    zq = ghi + _mm(oh, cbl_ref[...])
    zq_ref[0] = zq.T

    r = _dec_layer([ghi], d1a_ref[...], d1b_ref[...], bd1_ref[...])
    r = _dec_layer(r, d2a_ref[...], d2b_ref[...], bd2_ref[...])
    r = _dec_layer(r, d3a_ref[...], d3b_ref[...], bd3_ref[...])
    wot = wo_ref[...]
    bo = bo_ref[...]
    for j in range(8):
        rec_ref[0, j] = _mm(r[j], wot) + bo


def _fused(xp, wpt, bp, w1, b1, w2, b2, w3, b3, wqt, bq, cbt, cbn2, cbh, cb,
           d1a, d1b, bd1, d2a, d2b, bd2, d3a, d3b, bd3, wot, bo,
           interpret=False):
    B, _, U, F = xp.shape
    H = wpt.shape[1]
    D, K = cbt.shape
    full = lambda *shape: pl.BlockSpec(shape, lambda b: (0,) * len(shape))
    return pl.pallas_call(
        _body,
        grid=(B,),
        in_specs=[
            pl.BlockSpec((1, 8, U, F), lambda b: (b, 0, 0, 0)),
            full(F, H), full(1, H),
            full(3 * H, H), full(1, H),
            full(3 * H, H), full(1, H),
            full(3 * H, H), full(1, H),
            full(H, D), full(1, D),
            full(D, K), full(1, K),
            full(K, D), full(K, D),
            full(D, H), full(2 * D, H), full(1, H),
            full(H, H), full(2 * H, H), full(1, H),
            full(H, H), full(2 * H, H), full(1, H),
            full(H, F), full(1, F),
        ],
        out_specs=[
            pl.BlockSpec((1, 8, U, F), lambda b: (b, 0, 0, 0)),
            pl.BlockSpec((1, D, U), lambda b: (b, 0, 0)),
            pl.BlockSpec((1, D, U), lambda b: (b, 0, 0)),
            pl.BlockSpec((1, 1, U), lambda b: (b, 0, 0)),
        ],
        out_shape=[
            jax.ShapeDtypeStruct((B, 8, U, F), jnp.float32),
            jax.ShapeDtypeStruct((B, D, U), jnp.float32),
            jax.ShapeDtypeStruct((B, D, U), jnp.float32),
            jax.ShapeDtypeStruct((B, 1, U), jnp.int32),
        ],
        scratch_shapes=[pltpu.VMEM((K, D), _BF)],
        interpret=interpret,
    )(xp, wpt, bp, w1, b1, w2, b2, w3, b3, wqt, bq, cbt, cbn2, cbh, cb,
      d1a, d1b, bd1, d2a, d2b, bd2, d3a, d3b, bd3, wot, bo)


def kernel(x, Wp, bp, We1, be1, We2, be2, We3, be3, Wq, bq, codebook,
           Wd1, bd1, Wd2, bd2, Wd3, bd3, Wo, bo):
    B, F, T = x.shape
    H = Wp.shape[0]
    K, D = codebook.shape

    wpt = Wp.T.astype(_BF)
    enc_w = lambda w: w.transpose(2, 1, 0).reshape(3 * H, H).astype(_BF)
    wqt = Wq[:, :, 0].T.astype(_BF)
    cbt = codebook.T.astype(_BF)
    cbn2 = jnp.sum(codebook ** 2, axis=1).reshape(1, K)
    cbh = codebook.astype(_BF)
    dec_w1 = lambda w: w[:, :, 1].astype(_BF)
    dec_w20 = lambda w: jnp.concatenate([w[:, :, 2], w[:, :, 0]], axis=0).astype(_BF)
    r1 = lambda v: v.reshape(1, -1)

    U = T // 8
    xp = x.reshape(B, F, U, 8).transpose(0, 3, 2, 1)
    rec, z, z_q, idx = _fused(
        xp, wpt, r1(bp), enc_w(We1), r1(be1), enc_w(We2), r1(be2),
        enc_w(We3), r1(be3), wqt, r1(bq), cbt, cbn2, cbh, codebook,
        dec_w1(Wd1), dec_w20(Wd1), r1(bd1),
        dec_w1(Wd2), dec_w20(Wd2), r1(bd2),
        dec_w1(Wd3), dec_w20(Wd3), r1(bd3),
        Wo.T.astype(_BF), r1(bo))
    recon = rec.transpose(0, 3, 2, 1).reshape(B, F, T)
    return (recon, z, z_q, idx.reshape(B * U))


# speedup vs baseline: 1.1150x; 1.0237x over previous
import jax
import jax.numpy as jnp
from jax import lax
from jax.experimental import pallas as pl
from jax.experimental.pallas import tpu as pltpu

_BF = jnp.bfloat16


def _mm(a, b):
    return lax.dot_general(a.astype(_BF), b, (((1,), (0,)), ((), ())),
                           preferred_element_type=jnp.float32)


def _mm_nt(a, b):
    return lax.dot_general(a.astype(_BF), b.astype(_BF), (((1,), (1,)), ((), ())),
                           preferred_element_type=jnp.float32)


def _shift_down(a):
    return jnp.concatenate([jnp.zeros((1, a.shape[1]), a.dtype), a[:-1]], axis=0)


def _shift_up(a):
    return jnp.concatenate([a[1:], jnp.zeros((1, a.shape[1]), a.dtype)], axis=0)


def _enc_layer(phases, w, b):
    m = len(phases)
    out = []
    for q in range(m // 2):
        lo = 2 * q - 1
        a = _shift_down(phases[m - 1]) if lo < 0 else phases[lo]
        cat = jnp.concatenate([a, phases[2 * q], phases[2 * q + 1]], axis=1)
        out.append(jax.nn.relu(_mm(cat, w) + b))
    return out


def _dec_layer(phases, w1, w20, b):
    m = len(phases)
    out = []
    for i in range(m):
        out.append(jax.nn.relu(_mm(phases[i], w1) + b))
        nxt = phases[i + 1] if i < m - 1 else _shift_up(phases[0])
        cat = jnp.concatenate([phases[i], nxt], axis=1)
        out.append(jax.nn.relu(_mm(cat, w20) + b))
    return out


def _body(x_ref, wp_ref, bp_ref, w1_ref, b1_ref, w2_ref, b2_ref,
          w3_ref, b3_ref, wq_ref, bq_ref, cbn2_ref,
          cb_ref, d1a_ref, d1b_ref, bd1_ref, d2a_ref, d2b_ref,
          bd2_ref, d3a_ref, d3b_ref, bd3_ref, wo_ref, bo_ref,
          rec_ref, z_ref, zq_ref, idx_ref, cbh_ref, cbl_ref,
          wpt_ref, wqt_ref):
    b = pl.program_id(0)

    @pl.when(b == 0)
    def _():
        cbh = cb_ref[...].astype(_BF)
        cbh_ref[...] = cbh
        cbl_ref[...] = (cb_ref[...] - cbh.astype(jnp.float32)).astype(_BF)
        wpt_ref[...] = wp_ref[...].T.astype(_BF)
        wqt_ref[...] = wq_ref[...].T.astype(_BF)

    x = x_ref[0]
    wpt = wpt_ref[...]
    bp = bp_ref[...]
    h = [_mm(x[p], wpt) + bp for p in range(8)]

    y = _enc_layer(h, w1_ref[...], b1_ref[...])
    y = _enc_layer(y, w2_ref[...], b2_ref[...])
    y = _enc_layer(y, w3_ref[...], b3_ref[...])
    z = _mm(y[0], wqt_ref[...]) + bq_ref[...]
    z_ref[0] = z.T

    zc = _mm_nt(z, cbh_ref[...])
    zn2 = jnp.sum(z * z, axis=1, keepdims=True)
    d2 = zn2 - 2.0 * zc + cbn2_ref[...]
    d2 = jnp.maximum(d2, 0.0)
    mn = jnp.min(d2, axis=1, keepdims=True)
    iota = lax.broadcasted_iota(jnp.int32, d2.shape, 1)
    idx = jnp.min(jnp.where(d2 == mn, iota, d2.shape[1]), axis=1)
    idx_ref[0, 0] = idx

    oh = (iota == idx[:, None]).astype(_BF)
    ghi = _mm(oh, cbh_ref[...])
    zq = ghi + _mm(oh, cbl_ref[...])
    zq_ref[0] = zq.T

    r = _dec_layer([ghi], d1a_ref[...], d1b_ref[...], bd1_ref[...])
    r = _dec_layer(r, d2a_ref[...], d2b_ref[...], bd2_ref[...])
    r = _dec_layer(r, d3a_ref[...], d3b_ref[...], bd3_ref[...])
    wot = wo_ref[...]
    bo = bo_ref[...]
    for j in range(8):
        rec_ref[0, j] = _mm(r[j], wot) + bo


def _fused(xp, wp, bp, w1, b1, w2, b2, w3, b3, wq, bq, cbn2, cb,
           d1a, d1b, bd1, d2a, d2b, bd2, d3a, d3b, bd3, wot, bo,
           interpret=False):
    B, _, U, F = xp.shape
    H = wp.shape[0]
    K, D = cb.shape
    full = lambda *shape: pl.BlockSpec(shape, lambda b: (0,) * len(shape))
    return pl.pallas_call(
        _body,
        grid=(B,),
        in_specs=[
            pl.BlockSpec((1, 8, U, F), lambda b: (b, 0, 0, 0)),
            full(H, F), full(1, H),
            full(3 * H, H), full(1, H),
            full(3 * H, H), full(1, H),
            full(3 * H, H), full(1, H),
            full(H, D), full(1, D),
            full(1, K),
            full(K, D),
            full(D, H), full(2 * D, H), full(1, H),
            full(H, H), full(2 * H, H), full(1, H),
            full(H, H), full(2 * H, H), full(1, H),
            full(H, F), full(1, F),
        ],
        out_specs=[
            pl.BlockSpec((1, 8, U, F), lambda b: (b, 0, 0, 0)),
            pl.BlockSpec((1, D, U), lambda b: (b, 0, 0)),
            pl.BlockSpec((1, D, U), lambda b: (b, 0, 0)),
            pl.BlockSpec((1, 1, U), lambda b: (b, 0, 0)),
        ],
        out_shape=[
            jax.ShapeDtypeStruct((B, 8, U, F), jnp.float32),
            jax.ShapeDtypeStruct((B, D, U), jnp.float32),
            jax.ShapeDtypeStruct((B, D, U), jnp.float32),
            jax.ShapeDtypeStruct((B, 1, U), jnp.int32),
        ],
        scratch_shapes=[
            pltpu.VMEM((K, D), _BF), pltpu.VMEM((K, D), _BF),
            pltpu.VMEM((F, H), _BF), pltpu.VMEM((H, D), _BF),
        ],
        interpret=interpret,
    )(xp, wp, bp, w1, b1, w2, b2, w3, b3, wq, bq, cbn2, cb,
      d1a, d1b, bd1, d2a, d2b, bd2, d3a, d3b, bd3, wot, bo)


def kernel(x, Wp, bp, We1, be1, We2, be2, We3, be3, Wq, bq, codebook,
           Wd1, bd1, Wd2, bd2, Wd3, bd3, Wo, bo):
    B, F, T = x.shape
    H = Wp.shape[0]
    K, D = codebook.shape

    enc_w = lambda w: w.transpose(2, 1, 0).reshape(3 * H, H).astype(_BF)
    cbn2 = jnp.sum(codebook ** 2, axis=1).reshape(1, K)
    dec_w1 = lambda w: w[:, :, 1].astype(_BF)
    dec_w20 = lambda w: jnp.concatenate([w[:, :, 2], w[:, :, 0]], axis=0).astype(_BF)
    r1 = lambda v: v.reshape(1, -1)

    U = T // 8
    xp = x.reshape(B, F, U, 8).transpose(0, 3, 2, 1)
    rec, z, z_q, idx = _fused(
        xp, Wp, r1(bp), enc_w(We1), r1(be1), enc_w(We2), r1(be2),
        enc_w(We3), r1(be3), Wq.reshape(H, H), r1(bq), cbn2, codebook,
        dec_w1(Wd1), dec_w20(Wd1), r1(bd1),
        dec_w1(Wd2), dec_w20(Wd2), r1(bd2),
        dec_w1(Wd3), dec_w20(Wd3), r1(bd3),
        Wo.T.astype(_BF), r1(bo))
    recon = rec.transpose(0, 3, 2, 1).reshape(B, F, T)
    return (recon, z, z_q, idx.reshape(B * U))


# speedup vs baseline: 1.1663x; 1.0461x over previous
import jax
import jax.numpy as jnp
from jax import lax
from jax.experimental import pallas as pl
from jax.experimental.pallas import tpu as pltpu

_BF = jnp.bfloat16


def _mm(a, b):
    return lax.dot_general(a.astype(_BF), b, (((1,), (0,)), ((), ())),
                           preferred_element_type=jnp.float32)


def _mm_nt(a, b):
    return lax.dot_general(a.astype(_BF), b.astype(_BF), (((1,), (1,)), ((), ())),
                           preferred_element_type=jnp.float32)


def _shift_down(a):
    return jnp.concatenate([jnp.zeros((1, a.shape[1]), a.dtype), a[:-1]], axis=0)


def _shift_up(a):
    return jnp.concatenate([a[1:], jnp.zeros((1, a.shape[1]), a.dtype)], axis=0)


def _enc_layer(phases, w, b):
    m = len(phases)
    out = []
    for q in range(m // 2):
        lo = 2 * q - 1
        a = _shift_down(phases[m - 1]) if lo < 0 else phases[lo]
        cat = jnp.concatenate([a, phases[2 * q], phases[2 * q + 1]], axis=1)
        out.append(jax.nn.relu(_mm(cat, w) + b))
    return out


def _dec_layer(phases, w1, w20, b):
    m = len(phases)
    out = []
    for i in range(m):
        out.append(jax.nn.relu(_mm(phases[i], w1) + b))
        nxt = phases[i + 1] if i < m - 1 else _shift_up(phases[0])
        cat = jnp.concatenate([phases[i], nxt], axis=1)
        out.append(jax.nn.relu(_mm(cat, w20) + b))
    return out


def _body(x_ref, wp_ref, bp_ref, we_ref, b1_ref, b2_ref,
          b3_ref, wq_ref, bq_ref, cbn2_ref,
          cb_ref, wda_ref, wdb_ref, bd1_ref,
          bd2_ref, bd3_ref, wo_ref, bo_ref,
          rec_ref, z_ref, zq_ref, idx_ref, cbh_ref, cbl_ref,
          wpt_ref, wqt_ref):
    b = pl.program_id(0)

    @pl.when(b == 0)
    def _():
        cbh = cb_ref[...].astype(_BF)
        cbh_ref[...] = cbh
        cbl_ref[...] = (cb_ref[...] - cbh.astype(jnp.float32)).astype(_BF)
        wpt_ref[...] = wp_ref[...].T.astype(_BF)
        wqt_ref[...] = wq_ref[...].T.astype(_BF)

    x = x_ref[0]
    wpt = wpt_ref[...]
    bp = bp_ref[...]
    h = [_mm(x[p], wpt) + bp for p in range(8)]

    y = _enc_layer(h, we_ref[0], b1_ref[...])
    y = _enc_layer(y, we_ref[1], b2_ref[...])
    y = _enc_layer(y, we_ref[2], b3_ref[...])
    z = _mm(y[0], wqt_ref[...]) + bq_ref[...]
    z_ref[0] = z.T

    zc = _mm_nt(z, cbh_ref[...])
    zn2 = jnp.sum(z * z, axis=1, keepdims=True)
    d2 = zn2 - 2.0 * zc + cbn2_ref[...]
    d2 = jnp.maximum(d2, 0.0)
    mn = jnp.min(d2, axis=1, keepdims=True)
    iota = lax.broadcasted_iota(jnp.int32, d2.shape, 1)
    idx = jnp.min(jnp.where(d2 == mn, iota, d2.shape[1]), axis=1)
    idx_ref[0, 0] = idx

    oh = (iota == idx[:, None]).astype(_BF)
    ghi = _mm(oh, cbh_ref[...])
    zq = ghi + _mm(oh, cbl_ref[...])
    zq_ref[0] = zq.T

    r = _dec_layer([ghi], wda_ref[0], wdb_ref[0], bd1_ref[...])
    r = _dec_layer(r, wda_ref[1], wdb_ref[1], bd2_ref[...])
    r = _dec_layer(r, wda_ref[2], wdb_ref[2], bd3_ref[...])
    wot = wo_ref[...]
    bo = bo_ref[...]
    for j in range(8):
        rec_ref[0, j] = _mm(r[j], wot) + bo


def _fused(xp, wp, bp, we, b1, b2, b3, wq, bq, cbn2, cb,
           wda, wdb, bd1, bd2, bd3, wot, bo,
           interpret=False):
    B, _, U, F = xp.shape
    H = wp.shape[0]
    K, D = cb.shape
    full = lambda *shape: pl.BlockSpec(shape, lambda b: (0,) * len(shape))
    return pl.pallas_call(
        _body,
        grid=(B,),
        in_specs=[
            pl.BlockSpec((1, 8, U, F), lambda b: (b, 0, 0, 0)),
            full(H, F), full(1, H),
            full(3, 3 * H, H), full(1, H), full(1, H), full(1, H),
            full(H, D), full(1, D),
            full(1, K),
            full(K, D),
            full(3, H, H), full(3, 2 * H, H),
            full(1, H), full(1, H), full(1, H),
            full(H, F), full(1, F),
        ],
        out_specs=[
            pl.BlockSpec((1, 8, U, F), lambda b: (b, 0, 0, 0)),
            pl.BlockSpec((1, D, U), lambda b: (b, 0, 0)),
            pl.BlockSpec((1, D, U), lambda b: (b, 0, 0)),
            pl.BlockSpec((1, 1, U), lambda b: (b, 0, 0)),
        ],
        out_shape=[
            jax.ShapeDtypeStruct((B, 8, U, F), jnp.float32),
            jax.ShapeDtypeStruct((B, D, U), jnp.float32),
            jax.ShapeDtypeStruct((B, D, U), jnp.float32),
            jax.ShapeDtypeStruct((B, 1, U), jnp.int32),
        ],
        scratch_shapes=[
            pltpu.VMEM((K, D), _BF), pltpu.VMEM((K, D), _BF),
            pltpu.VMEM((F, H), _BF), pltpu.VMEM((H, D), _BF),
        ],
        interpret=interpret,
    )(xp, wp, bp, we, b1, b2, b3, wq, bq, cbn2, cb,
      wda, wdb, bd1, bd2, bd3, wot, bo)


def kernel(x, Wp, bp, We1, be1, We2, be2, We3, be3, Wq, bq, codebook,
           Wd1, bd1, Wd2, bd2, Wd3, bd3, Wo, bo):
    B, F, T = x.shape
    H = Wp.shape[0]
    K, D = codebook.shape

    cbn2 = jnp.sum(codebook ** 2, axis=1).reshape(1, K)
    we = jnp.stack([We1, We2, We3])
    we = we.transpose(0, 3, 2, 1).reshape(3, 3 * H, H).astype(_BF)
    wd = jnp.stack([Wd1, Wd2, Wd3])
    wda = wd[:, :, :, 1].astype(_BF)
    wdb = jnp.concatenate([wd[:, :, :, 2], wd[:, :, :, 0]], axis=1).astype(_BF)
    r1 = lambda v: v.reshape(1, -1)

    U = T // 8
    xp = x.reshape(B, F, U, 8).transpose(0, 3, 2, 1)
    rec, z, z_q, idx = _fused(
        xp, Wp, r1(bp), we, r1(be1), r1(be2), r1(be3),
        Wq.reshape(H, H), r1(bq), cbn2, codebook,
        wda, wdb, r1(bd1), r1(bd2), r1(bd3),
        Wo.T.astype(_BF), r1(bo))
    recon = rec.transpose(0, 3, 2, 1).reshape(B, F, T)
    return (recon, z, z_q, idx.reshape(B * U))
